# baseline (speedup 1.0000x reference)
import functools

import jax
import jax.numpy as jnp
from jax import lax
from jax.experimental import pallas as pl
from jax.experimental.pallas import tpu as pltpu
from jax.experimental.pallas import tpu_sc as plsc

N = 10000
D = 256
E = 160000
NC = 2
NS = 16
NP = 10240
RPT = 640
EPAD = 163840

_MESH = plsc.VectorSubcoreMesh(core_axis_name="c", subcore_axis_name="s",
                               num_cores=NC, num_subcores=NS)
_SC_PARAMS = pltpu.CompilerParams(needs_layout_passes=False)

_Z16 = functools.partial(jnp.zeros, (16,), jnp.float32)


def _deg_body(epad_hbm, deg_hbm, acc_sh, zbuf, ones_v, idx_a, idx_b,
              sem_a, sem_b):
    c = lax.axis_index("c")
    s = lax.axis_index("s")
    wid = c * NS + s

    z16 = _Z16()
    o16 = jnp.ones((16,), jnp.float32)

    def _zb(i, carry):
        zbuf[pl.ds(i * 16, 16)] = z16
        return carry
    lax.fori_loop(0, RPT // 16, _zb, 0)
    for j in range(8):
        ones_v[pl.ds(j * 16, 16)] = o16
    pltpu.sync_copy(zbuf, acc_sh.at[pl.ds(s * RPT, RPT)])
    plsc.subcore_barrier()

    ebase = wid * 5120

    def _issue(k, idx, sem):
        pltpu.sync_copy(epad_hbm.at[1, pl.ds(ebase + k * 128, 128)], idx)
        pltpu.async_copy(ones_v, acc_sh.at[idx], sem, add=True)

    _issue(0, idx_a, sem_a)
    _issue(1, idx_b, sem_b)

    def _body(t, carry):
        k = 2 * t
        pltpu.make_async_copy(ones_v, acc_sh.at[idx_a], sem_a).wait()

        @pl.when(k + 2 < 40)
        def _():
            _issue(k + 2, idx_a, sem_a)
        pltpu.make_async_copy(ones_v, acc_sh.at[idx_b], sem_b).wait()

        @pl.when(k + 3 < 40)
        def _():
            _issue(k + 3, idx_b, sem_b)
        return carry
    lax.fori_loop(0, 20, _body, 0)
    plsc.subcore_barrier()
    pltpu.sync_copy(acc_sh.at[pl.ds(s * RPT, RPT)],
                    deg_hbm.at[c, pl.ds(s * RPT, RPT)])


_deg_call = functools.partial(
    pl.kernel,
    out_type=jax.ShapeDtypeStruct((NC, NP), jnp.float32),
    mesh=_MESH,
    compiler_params=_SC_PARAMS,
    scratch_types=[
        pltpu.VMEM_SHARED((NP,), jnp.float32),
        pltpu.VMEM((RPT,), jnp.float32),
        pltpu.VMEM((128,), jnp.float32),
        pltpu.VMEM((128,), jnp.int32),
        pltpu.VMEM((128,), jnp.int32),
        pltpu.SemaphoreType.DMA,
        pltpu.SemaphoreType.DMA,
    ],
)(_deg_body)


def _linear_body(x_ref, w_ref, b_ref, degp_ref, xh_ref, d_ref, d2_ref):
    deg = degp_ref[0, :, 0] + degp_ref[1, :, 0]
    dv = jnp.where(deg > 0, lax.rsqrt(deg), 0.0)
    res = lax.dot_general(x_ref[...], w_ref[...], (((1,), (1,)), ((), ())),
                          preferred_element_type=jnp.float32)
    xh_ref[0] = (res + b_ref[0]) * dv[:, None]
    d_ref[...] = dv[:, None]
    d2_ref[...] = (dv * dv)[:, None]


def _linear_call(x, w, b2, degp):
    return pl.pallas_call(
        _linear_body,
        grid=(16, 2),
        in_specs=[
            pl.BlockSpec((RPT, 256), lambda i, h: (i, 0)),
            pl.BlockSpec((128, 256), lambda i, h: (h, 0)),
            pl.BlockSpec((1, 1, 128), lambda i, h: (h, 0, 0)),
            pl.BlockSpec((2, RPT, 1), lambda i, h: (0, i, 0)),
        ],
        out_specs=[
            pl.BlockSpec((1, RPT, 128), lambda i, h: (h, i, 0)),
            pl.BlockSpec((RPT, 1), lambda i, h: (i, 0)),
            pl.BlockSpec((RPT, 1), lambda i, h: (i, 0)),
        ],
        out_shape=[
            jax.ShapeDtypeStruct((2, NP, 128), jnp.float32),
            jax.ShapeDtypeStruct((NP, 1), jnp.float32),
            jax.ShapeDtypeStruct((NP, 1), jnp.float32),
        ],
    )(x, w, b2, degp)


def _conv_body(epad_hbm, zin_hbm, zout_hbm, acc_sh,
               e2_a, e2_b, src_a, dst_a, src_b, dst_b, rows_a, rows_b,
               gsem_a, gsem_b, ssem_a, ssem_b):
    c = lax.axis_index("c")
    s = lax.axis_index("s")

    z16 = _Z16()

    def _zb(i, carry):
        for j in range(8):
            rows_a[i, pl.ds(j * 16, 16)] = z16
        return carry
    lax.fori_loop(0, 128, _zb, 0)
    for q in range(5):
        pltpu.sync_copy(rows_a, acc_sh.at[pl.ds(s * RPT + q * 128, 128)])
    plsc.subcore_barrier()

    ebase = s * 10240
    roff = c * NP

    def _load_idx(k, e2, srcv, dstv):
        pltpu.sync_copy(epad_hbm.at[:, pl.ds(ebase + k * 128, 128)], e2)
        for j in range(8):
            sl = pl.ds(j * 16, 16)
            srcv[sl] = e2[0, sl] + roff
            dstv[sl] = e2[1, sl]

    def _half(k, e2, srcv, dstv, rows, gsem, ssem):
        pltpu.make_async_copy(zin_hbm.at[srcv], rows, gsem).wait()
        pltpu.async_copy(rows, acc_sh.at[dstv], ssem, add=True)
        pltpu.make_async_copy(rows, acc_sh.at[dstv], ssem).wait()

        @pl.when(k + 2 < 80)
        def _():
            _load_idx(k + 2, e2, srcv, dstv)
            pltpu.async_copy(zin_hbm.at[srcv], rows, gsem)

    _load_idx(0, e2_a, src_a, dst_a)
    pltpu.async_copy(zin_hbm.at[src_a], rows_a, gsem_a)
    _load_idx(1, e2_b, src_b, dst_b)
    pltpu.async_copy(zin_hbm.at[src_b], rows_b, gsem_b)

    def _body(t, carry):
        _half(2 * t, e2_a, src_a, dst_a, rows_a, gsem_a, ssem_a)
        _half(2 * t + 1, e2_b, src_b, dst_b, rows_b, gsem_b, ssem_b)
        return carry
    lax.fori_loop(0, 40, _body, 0)
    plsc.subcore_barrier()
    pltpu.sync_copy(acc_sh.at[pl.ds(s * RPT, RPT)],
                    zout_hbm.at[pl.ds(c * NP + s * RPT, RPT)])


def _conv_call(eidx, zin):
    return pl.kernel(
        _conv_body,
        out_type=jax.ShapeDtypeStruct((2 * NP, 128), jnp.float32),
        mesh=_MESH,
        compiler_params=_SC_PARAMS,
        scratch_types=[
            pltpu.VMEM_SHARED((NP, 128), jnp.float32),
            pltpu.VMEM((2, 128), jnp.int32),
            pltpu.VMEM((2, 128), jnp.int32),
            pltpu.VMEM((128,), jnp.int32),
            pltpu.VMEM((128,), jnp.int32),
            pltpu.VMEM((128,), jnp.int32),
            pltpu.VMEM((128,), jnp.int32),
            pltpu.VMEM((128, 128), jnp.float32),
            pltpu.VMEM((128, 128), jnp.float32),
            pltpu.SemaphoreType.DMA,
            pltpu.SemaphoreType.DMA,
            pltpu.SemaphoreType.DMA,
            pltpu.SemaphoreType.DMA,
        ],
    )(eidx, zin)


def _scale_body(z_ref, d2_ref, o_ref):
    o_ref[...] = z_ref[...] * d2_ref[...]


def _scale_call(z2, d2p):
    return pl.pallas_call(
        _scale_body,
        grid=(32,),
        in_specs=[
            pl.BlockSpec((RPT, 128), lambda j: (j, 0)),
            pl.BlockSpec((RPT, 1), lambda j: (j % 16, 0)),
        ],
        out_specs=pl.BlockSpec((RPT, 128), lambda j: (j, 0)),
        out_shape=jax.ShapeDtypeStruct((2 * NP, 128), jnp.float32),
    )(z2, d2p)


def _dot_body(ei_hbm, z4_hbm, out_hbm, e2_a, e2_b,
              ai_a, bi_a, ai_b, bi_b, rows_aa, rows_ab, rows_ba, rows_bb,
              outbuf, ga1, ga2, gb1, gb2):
    c = lax.axis_index("c")
    s = lax.axis_index("s")
    ebase = s * 10240
    roff = c * NP

    lane = lax.iota(jnp.int32, 16)

    def _issue(k, e2, ai, bi, ra, rb, s1, s2):
        pltpu.sync_copy(ei_hbm.at[:, pl.ds(ebase + k * 128, 128)], e2)
        for j in range(8):
            sl = pl.ds(j * 16, 16)
            ai[sl] = e2[0, sl] + roff
            bi[sl] = e2[1, sl] + roff
        pltpu.async_copy(z4_hbm.at[ai], ra, s1)
        pltpu.async_copy(z4_hbm.at[bi], rb, s2)

    def _compute(k, e2, ai, bi, ra, rb, s1, s2):
        pltpu.make_async_copy(z4_hbm.at[ai], ra, s1).wait()
        pltpu.make_async_copy(z4_hbm.at[bi], rb, s2).wait()
        for g in range(8):
            rowidx = g * 16 + lane

            def _cj(jo, accs):
                j0 = jo * 8
                out = []
                for jj in range(8):
                    cj = jnp.full((16,), jj, jnp.int32) + j0
                    av = plsc.load_gather(ra, [rowidx, cj])
                    bv = plsc.load_gather(rb, [rowidx, cj])
                    out.append(accs[jj] + av * bv)
                return tuple(out)
            accs = lax.fori_loop(0, 16, _cj, (_Z16(),) * 8)
            res = (((accs[0] + accs[1]) + (accs[2] + accs[3]))
                   + ((accs[4] + accs[5]) + (accs[6] + accs[7])))
            sl = pl.ds(g * 16, 16)
            outbuf[sl] = res
        pltpu.sync_copy(outbuf, out_hbm.at[c, pl.ds(ebase + k * 128, 128)])

    _issue(0, e2_a, ai_a, bi_a, rows_aa, rows_ab, ga1, ga2)
    _issue(1, e2_b, ai_b, bi_b, rows_ba, rows_bb, gb1, gb2)

    def _body(t, carry):
        k = 2 * t
        _compute(k, e2_a, ai_a, bi_a, rows_aa, rows_ab, ga1, ga2)

        @pl.when(k + 2 < 80)
        def _():
            _issue(k + 2, e2_a, ai_a, bi_a, rows_aa, rows_ab, ga1, ga2)
        _compute(k + 1, e2_b, ai_b, bi_b, rows_ba, rows_bb, gb1, gb2)

        @pl.when(k + 3 < 80)
        def _():
            _issue(k + 3, e2_b, ai_b, bi_b, rows_ba, rows_bb, gb1, gb2)
        return carry
    lax.fori_loop(0, 40, _body, 0)


_dot_call = functools.partial(
    pl.kernel,
    out_type=jax.ShapeDtypeStruct((NC, EPAD), jnp.float32),
    mesh=_MESH,
    compiler_params=_SC_PARAMS,
    scratch_types=[
        pltpu.VMEM((2, 128), jnp.int32),
        pltpu.VMEM((2, 128), jnp.int32),
        pltpu.VMEM((128,), jnp.int32),
        pltpu.VMEM((128,), jnp.int32),
        pltpu.VMEM((128,), jnp.int32),
        pltpu.VMEM((128,), jnp.int32),
        pltpu.VMEM((128, 128), jnp.float32),
        pltpu.VMEM((128, 128), jnp.float32),
        pltpu.VMEM((128, 128), jnp.float32),
        pltpu.VMEM((128, 128), jnp.float32),
        pltpu.VMEM((128,), jnp.float32),
        pltpu.SemaphoreType.DMA,
        pltpu.SemaphoreType.DMA,
        pltpu.SemaphoreType.DMA,
        pltpu.SemaphoreType.DMA,
    ],
)(_dot_body)


def _comb_body(p_ref, o_ref):
    o_ref[...] = p_ref[0, :] + p_ref[1, :]


def _comb_call(partials):
    return pl.pallas_call(
        _comb_body,
        grid=(80,),
        in_specs=[pl.BlockSpec((2, 2048), lambda j: (0, j))],
        out_specs=pl.BlockSpec((2048,), lambda j: (j,)),
        out_shape=jax.ShapeDtypeStruct((EPAD,), jnp.float32),
    )(partials)


def kernel(x_input, edge_index_input, pos_edge_index, W, b):
    padv = (N + jnp.arange(EPAD - E, dtype=jnp.int32) % (NP - N))
    epad = jnp.concatenate(
        [pos_edge_index, jnp.stack([padv, padv])], axis=1)
    eipad = jnp.concatenate(
        [edge_index_input, jnp.stack([padv, padv])], axis=1)
    xp = jnp.pad(x_input, ((0, NP - N), (0, 0)))
    b2 = b.reshape(2, 1, 128)

    degp = _deg_call(epad)
    x1h3, dcol, d2col = _linear_call(xp, W, b2, degp.reshape(2, NP, 1))
    x1h = x1h3.reshape(2 * NP, 128)

    z2 = _conv_call(epad, x1h)
    z3 = _scale_call(z2, d2col)
    z4 = _conv_call(epad, z3)
    x3h = _scale_call(z4, dcol)
    partials = _dot_call(eipad, x3h)
    logits = _comb_call(partials)
    return logits[:E]


# speedup vs baseline: 2.9532x; 2.9532x over previous
import functools

import jax
import jax.numpy as jnp
from jax import lax
from jax.experimental import pallas as pl
from jax.experimental.pallas import tpu as pltpu
from jax.experimental.pallas import tpu_sc as plsc

N = 10000
D = 256
E = 160000
NC = 2
NS = 16
NP = 10240
RPT = 640
EPAD = 163840

_MESH = plsc.VectorSubcoreMesh(core_axis_name="c", subcore_axis_name="s",
                               num_cores=NC, num_subcores=NS)
_SC_PARAMS = pltpu.CompilerParams(needs_layout_passes=False)

_Z16 = functools.partial(jnp.zeros, (16,), jnp.float32)


def _deg_body(epad_hbm, deg_hbm, acc_sh, zbuf, ones_v, idx_a, idx_b,
              sem_a, sem_b):
    c = lax.axis_index("c")
    s = lax.axis_index("s")
    wid = c * NS + s

    z16 = _Z16()
    o16 = jnp.ones((16,), jnp.float32)

    def _zb(i, carry):
        zbuf[pl.ds(i * 16, 16)] = z16
        return carry
    lax.fori_loop(0, RPT // 16, _zb, 0)
    for j in range(8):
        ones_v[pl.ds(j * 16, 16)] = o16
    pltpu.sync_copy(zbuf, acc_sh.at[pl.ds(s * RPT, RPT)])
    plsc.subcore_barrier()

    ebase = wid * 5120

    def _issue(k, idx, sem):
        pltpu.sync_copy(epad_hbm.at[1, pl.ds(ebase + k * 128, 128)], idx)
        pltpu.async_copy(ones_v, acc_sh.at[idx], sem, add=True)

    _issue(0, idx_a, sem_a)
    _issue(1, idx_b, sem_b)

    def _body(t, carry):
        k = 2 * t
        pltpu.make_async_copy(ones_v, acc_sh.at[idx_a], sem_a).wait()

        @pl.when(k + 2 < 40)
        def _():
            _issue(k + 2, idx_a, sem_a)
        pltpu.make_async_copy(ones_v, acc_sh.at[idx_b], sem_b).wait()

        @pl.when(k + 3 < 40)
        def _():
            _issue(k + 3, idx_b, sem_b)
        return carry
    lax.fori_loop(0, 20, _body, 0)
    plsc.subcore_barrier()
    pltpu.sync_copy(acc_sh.at[pl.ds(s * RPT, RPT)],
                    deg_hbm.at[c, pl.ds(s * RPT, RPT)])


_deg_call = functools.partial(
    pl.kernel,
    out_type=jax.ShapeDtypeStruct((NC, NP), jnp.float32),
    mesh=_MESH,
    compiler_params=_SC_PARAMS,
    scratch_types=[
        pltpu.VMEM_SHARED((NP,), jnp.float32),
        pltpu.VMEM((RPT,), jnp.float32),
        pltpu.VMEM((128,), jnp.float32),
        pltpu.VMEM((128,), jnp.int32),
        pltpu.VMEM((128,), jnp.int32),
        pltpu.SemaphoreType.DMA,
        pltpu.SemaphoreType.DMA,
    ],
)(_deg_body)


def _linear_body(x_ref, w_ref, b_ref, degp_ref, xh_ref, d_ref, d2_ref):
    deg = degp_ref[0, :, 0] + degp_ref[1, :, 0]
    dv = jnp.where(deg > 0, lax.rsqrt(deg), 0.0)
    res = lax.dot_general(x_ref[...], w_ref[...], (((1,), (1,)), ((), ())),
                          preferred_element_type=jnp.float32)
    xh_ref[0] = (res + b_ref[0]) * dv[:, None]
    d_ref[...] = dv[:, None]
    d2_ref[...] = (dv * dv)[:, None]


def _linear_call(x, w, b2, degp):
    return pl.pallas_call(
        _linear_body,
        grid=(16, 2),
        in_specs=[
            pl.BlockSpec((RPT, 256), lambda i, h: (i, 0)),
            pl.BlockSpec((128, 256), lambda i, h: (h, 0)),
            pl.BlockSpec((1, 1, 128), lambda i, h: (h, 0, 0)),
            pl.BlockSpec((2, RPT, 1), lambda i, h: (0, i, 0)),
        ],
        out_specs=[
            pl.BlockSpec((1, RPT, 128), lambda i, h: (h, i, 0)),
            pl.BlockSpec((RPT, 1), lambda i, h: (i, 0)),
            pl.BlockSpec((RPT, 1), lambda i, h: (i, 0)),
        ],
        out_shape=[
            jax.ShapeDtypeStruct((2, NP, 128), jnp.float32),
            jax.ShapeDtypeStruct((NP, 1), jnp.float32),
            jax.ShapeDtypeStruct((NP, 1), jnp.float32),
        ],
    )(x, w, b2, degp)


def _conv_body(epad_hbm, zin_hbm, zout_hbm, acc_sh,
               e2_a, e2_b, src_a, dst_a, src_b, dst_b, rows_a, rows_b,
               gsem_a, gsem_b, ssem_a, ssem_b):
    c = lax.axis_index("c")
    s = lax.axis_index("s")

    z16 = _Z16()

    def _zb(i, carry):
        for j in range(8):
            rows_a[i, pl.ds(j * 16, 16)] = z16
        return carry
    lax.fori_loop(0, 128, _zb, 0)
    for q in range(5):
        pltpu.sync_copy(rows_a, acc_sh.at[pl.ds(s * RPT + q * 128, 128)])
    plsc.subcore_barrier()

    ebase = s * 10240
    roff = c * NP

    def _load_idx(k, e2, srcv, dstv):
        pltpu.sync_copy(epad_hbm.at[:, pl.ds(ebase + k * 128, 128)], e2)
        for j in range(8):
            sl = pl.ds(j * 16, 16)
            srcv[sl] = e2[0, sl] + roff
            dstv[sl] = e2[1, sl]

    def _half(k, e2, srcv, dstv, rows, gsem, ssem):
        pltpu.make_async_copy(zin_hbm.at[srcv], rows, gsem).wait()
        pltpu.async_copy(rows, acc_sh.at[dstv], ssem, add=True)
        pltpu.make_async_copy(rows, acc_sh.at[dstv], ssem).wait()

        @pl.when(k + 2 < 80)
        def _():
            _load_idx(k + 2, e2, srcv, dstv)
            pltpu.async_copy(zin_hbm.at[srcv], rows, gsem)

    _load_idx(0, e2_a, src_a, dst_a)
    pltpu.async_copy(zin_hbm.at[src_a], rows_a, gsem_a)
    _load_idx(1, e2_b, src_b, dst_b)
    pltpu.async_copy(zin_hbm.at[src_b], rows_b, gsem_b)

    def _body(t, carry):
        _half(2 * t, e2_a, src_a, dst_a, rows_a, gsem_a, ssem_a)
        _half(2 * t + 1, e2_b, src_b, dst_b, rows_b, gsem_b, ssem_b)
        return carry
    lax.fori_loop(0, 40, _body, 0)
    plsc.subcore_barrier()
    pltpu.sync_copy(acc_sh.at[pl.ds(s * RPT, RPT)],
                    zout_hbm.at[pl.ds(c * NP + s * RPT, RPT)])


def _conv_call(eidx, zin):
    return pl.kernel(
        _conv_body,
        out_type=jax.ShapeDtypeStruct((2 * NP, 128), jnp.float32),
        mesh=_MESH,
        compiler_params=_SC_PARAMS,
        scratch_types=[
            pltpu.VMEM_SHARED((NP, 128), jnp.float32),
            pltpu.VMEM((2, 128), jnp.int32),
            pltpu.VMEM((2, 128), jnp.int32),
            pltpu.VMEM((128,), jnp.int32),
            pltpu.VMEM((128,), jnp.int32),
            pltpu.VMEM((128,), jnp.int32),
            pltpu.VMEM((128,), jnp.int32),
            pltpu.VMEM((128, 128), jnp.float32),
            pltpu.VMEM((128, 128), jnp.float32),
            pltpu.SemaphoreType.DMA,
            pltpu.SemaphoreType.DMA,
            pltpu.SemaphoreType.DMA,
            pltpu.SemaphoreType.DMA,
        ],
    )(eidx, zin)


def _scale_body(z_ref, d2_ref, o_ref):
    o_ref[...] = z_ref[...] * d2_ref[...]


def _scale_call(z2, d2p):
    return pl.pallas_call(
        _scale_body,
        grid=(32,),
        in_specs=[
            pl.BlockSpec((RPT, 128), lambda j: (j, 0)),
            pl.BlockSpec((RPT, 1), lambda j: (j % 16, 0)),
        ],
        out_specs=pl.BlockSpec((RPT, 128), lambda j: (j, 0)),
        out_shape=jax.ShapeDtypeStruct((2 * NP, 128), jnp.float32),
    )(z2, d2p)


def _dot_body(ei_hbm, z4_hbm, out_hbm, e2_a, e2_b,
              ai_a, bi_a, ai_b, bi_b, rows_aa, rows_ab, rows_ba, rows_bb,
              outbuf, ga1, ga2, gb1, gb2):
    c = lax.axis_index("c")
    s = lax.axis_index("s")
    ebase = s * 10240
    roff = c * NP

    lane = lax.iota(jnp.int32, 16)

    def _issue(k, e2, ai, bi, ra, rb, s1, s2):
        pltpu.sync_copy(ei_hbm.at[:, pl.ds(ebase + k * 128, 128)], e2)
        for j in range(8):
            sl = pl.ds(j * 16, 16)
            ai[sl] = e2[0, sl] + roff
            bi[sl] = e2[1, sl] + roff
        pltpu.async_copy(z4_hbm.at[ai], ra, s1)
        pltpu.async_copy(z4_hbm.at[bi], rb, s2)

    def _compute(k, e2, ai, bi, ra, rb, s1, s2):
        pltpu.make_async_copy(z4_hbm.at[ai], ra, s1).wait()
        pltpu.make_async_copy(z4_hbm.at[bi], rb, s2).wait()
        for g in range(8):
            rowidx = g * 16 + lane

            def _cj(jo, accs):
                j0 = jo * 8
                out = []
                for jj in range(8):
                    cj = (jnp.full((16,), jj, jnp.int32) + j0 + lane) & 127
                    av = plsc.load_gather(ra, [rowidx, cj])
                    bv = plsc.load_gather(rb, [rowidx, cj])
                    out.append(accs[jj] + av * bv)
                return tuple(out)
            accs = lax.fori_loop(0, 16, _cj, (_Z16(),) * 8)
            res = (((accs[0] + accs[1]) + (accs[2] + accs[3]))
                   + ((accs[4] + accs[5]) + (accs[6] + accs[7])))
            sl = pl.ds(g * 16, 16)
            outbuf[sl] = res
        pltpu.sync_copy(outbuf, out_hbm.at[c, pl.ds(ebase + k * 128, 128)])

    _issue(0, e2_a, ai_a, bi_a, rows_aa, rows_ab, ga1, ga2)
    _issue(1, e2_b, ai_b, bi_b, rows_ba, rows_bb, gb1, gb2)

    def _body(t, carry):
        k = 2 * t
        _compute(k, e2_a, ai_a, bi_a, rows_aa, rows_ab, ga1, ga2)

        @pl.when(k + 2 < 80)
        def _():
            _issue(k + 2, e2_a, ai_a, bi_a, rows_aa, rows_ab, ga1, ga2)
        _compute(k + 1, e2_b, ai_b, bi_b, rows_ba, rows_bb, gb1, gb2)

        @pl.when(k + 3 < 80)
        def _():
            _issue(k + 3, e2_b, ai_b, bi_b, rows_ba, rows_bb, gb1, gb2)
        return carry
    lax.fori_loop(0, 40, _body, 0)


_dot_call = functools.partial(
    pl.kernel,
    out_type=jax.ShapeDtypeStruct((NC, EPAD), jnp.float32),
    mesh=_MESH,
    compiler_params=_SC_PARAMS,
    scratch_types=[
        pltpu.VMEM((2, 128), jnp.int32),
        pltpu.VMEM((2, 128), jnp.int32),
        pltpu.VMEM((128,), jnp.int32),
        pltpu.VMEM((128,), jnp.int32),
        pltpu.VMEM((128,), jnp.int32),
        pltpu.VMEM((128,), jnp.int32),
        pltpu.VMEM((128, 128), jnp.float32),
        pltpu.VMEM((128, 128), jnp.float32),
        pltpu.VMEM((128, 128), jnp.float32),
        pltpu.VMEM((128, 128), jnp.float32),
        pltpu.VMEM((128,), jnp.float32),
        pltpu.SemaphoreType.DMA,
        pltpu.SemaphoreType.DMA,
        pltpu.SemaphoreType.DMA,
        pltpu.SemaphoreType.DMA,
    ],
)(_dot_body)


def _comb_body(p_ref, o_ref):
    o_ref[...] = p_ref[0, :] + p_ref[1, :]


def _comb_call(partials):
    return pl.pallas_call(
        _comb_body,
        grid=(80,),
        in_specs=[pl.BlockSpec((2, 2048), lambda j: (0, j))],
        out_specs=pl.BlockSpec((2048,), lambda j: (j,)),
        out_shape=jax.ShapeDtypeStruct((EPAD,), jnp.float32),
    )(partials)


def kernel(x_input, edge_index_input, pos_edge_index, W, b):
    padv = (N + jnp.arange(EPAD - E, dtype=jnp.int32) % (NP - N))
    epad = jnp.concatenate(
        [pos_edge_index, jnp.stack([padv, padv])], axis=1)
    eipad = jnp.concatenate(
        [edge_index_input, jnp.stack([padv, padv])], axis=1)
    xp = jnp.pad(x_input, ((0, NP - N), (0, 0)))
    b2 = b.reshape(2, 1, 128)

    degp = _deg_call(epad)
    x1h3, dcol, d2col = _linear_call(xp, W, b2, degp.reshape(2, NP, 1))
    x1h = x1h3.reshape(2 * NP, 128)

    z2 = _conv_call(epad, x1h)
    z3 = _scale_call(z2, d2col)
    z4 = _conv_call(epad, z3)
    x3h = _scale_call(z4, dcol)
    partials = _dot_call(eipad, x3h)
    logits = _comb_call(partials)
    return logits[:E]


# speedup vs baseline: 3.0451x; 1.0311x over previous
import functools

import jax
import jax.numpy as jnp
from jax import lax
from jax.experimental import pallas as pl
from jax.experimental.pallas import tpu as pltpu
from jax.experimental.pallas import tpu_sc as plsc

N = 10000
D = 256
E = 160000
NC = 2
NS = 16
NP = 10240
RPT = 640
EPAD = 163840

_MESH = plsc.VectorSubcoreMesh(core_axis_name="c", subcore_axis_name="s",
                               num_cores=NC, num_subcores=NS)
_SC_PARAMS = pltpu.CompilerParams(needs_layout_passes=False)

_Z16 = functools.partial(jnp.zeros, (16,), jnp.float32)


def _deg_body(epad_hbm, deg_hbm, acc_sh, zbuf, ones_v, idx_a, idx_b,
              sem_a, sem_b):
    c = lax.axis_index("c")
    s = lax.axis_index("s")
    wid = c * NS + s

    z16 = _Z16()
    o16 = jnp.ones((16,), jnp.float32)

    def _zb(i, carry):
        zbuf[pl.ds(i * 16, 16)] = z16
        return carry
    lax.fori_loop(0, RPT // 16, _zb, 0)
    for j in range(8):
        ones_v[pl.ds(j * 16, 16)] = o16
    pltpu.sync_copy(zbuf, acc_sh.at[pl.ds(s * RPT, RPT)])
    plsc.subcore_barrier()

    ebase = wid * 5120

    def _issue(k, idx, sem):
        pltpu.sync_copy(epad_hbm.at[1, pl.ds(ebase + k * 128, 128)], idx)
        pltpu.async_copy(ones_v, acc_sh.at[idx], sem, add=True)

    _issue(0, idx_a, sem_a)
    _issue(1, idx_b, sem_b)

    def _body(t, carry):
        k = 2 * t
        pltpu.make_async_copy(ones_v, acc_sh.at[idx_a], sem_a).wait()

        @pl.when(k + 2 < 40)
        def _():
            _issue(k + 2, idx_a, sem_a)
        pltpu.make_async_copy(ones_v, acc_sh.at[idx_b], sem_b).wait()

        @pl.when(k + 3 < 40)
        def _():
            _issue(k + 3, idx_b, sem_b)
        return carry
    lax.fori_loop(0, 20, _body, 0)
    plsc.subcore_barrier()
    pltpu.sync_copy(acc_sh.at[pl.ds(s * RPT, RPT)],
                    deg_hbm.at[c, pl.ds(s * RPT, RPT)])


_deg_call = functools.partial(
    pl.kernel,
    out_type=jax.ShapeDtypeStruct((NC, NP), jnp.float32),
    mesh=_MESH,
    compiler_params=_SC_PARAMS,
    scratch_types=[
        pltpu.VMEM_SHARED((NP,), jnp.float32),
        pltpu.VMEM((RPT,), jnp.float32),
        pltpu.VMEM((128,), jnp.float32),
        pltpu.VMEM((128,), jnp.int32),
        pltpu.VMEM((128,), jnp.int32),
        pltpu.SemaphoreType.DMA,
        pltpu.SemaphoreType.DMA,
    ],
)(_deg_body)


def _linear_body(x_ref, w_ref, b_ref, degp_ref, xh_ref, d_ref, d2_ref):
    deg = degp_ref[0, :, 0] + degp_ref[1, :, 0]
    dv = jnp.where(deg > 0, lax.rsqrt(deg), 0.0)
    res = lax.dot_general(x_ref[...], w_ref[...], (((1,), (1,)), ((), ())),
                          preferred_element_type=jnp.float32)
    xh_ref[0] = (res + b_ref[0]) * dv[:, None]
    d_ref[...] = dv[:, None]
    d2_ref[...] = (dv * dv)[:, None]


def _linear_call(x, w, b2, degp):
    return pl.pallas_call(
        _linear_body,
        grid=(16, 2),
        in_specs=[
            pl.BlockSpec((RPT, 256), lambda i, h: (i, 0)),
            pl.BlockSpec((128, 256), lambda i, h: (h, 0)),
            pl.BlockSpec((1, 1, 128), lambda i, h: (h, 0, 0)),
            pl.BlockSpec((2, RPT, 1), lambda i, h: (0, i, 0)),
        ],
        out_specs=[
            pl.BlockSpec((1, RPT, 128), lambda i, h: (h, i, 0)),
            pl.BlockSpec((RPT, 1), lambda i, h: (i, 0)),
            pl.BlockSpec((RPT, 1), lambda i, h: (i, 0)),
        ],
        out_shape=[
            jax.ShapeDtypeStruct((2, NP, 128), jnp.float32),
            jax.ShapeDtypeStruct((NP, 1), jnp.float32),
            jax.ShapeDtypeStruct((NP, 1), jnp.float32),
        ],
    )(x, w, b2, degp)


def _conv_body(epad_hbm, zin_hbm, zout_hbm, acc_sh,
               e2_a, e2_b, src_a, dst_a, src_b, dst_b, rows_a, rows_b,
               gsem_a, gsem_b, ssem_a, ssem_b):
    c = lax.axis_index("c")
    s = lax.axis_index("s")

    z16 = _Z16()

    def _zb(i, carry):
        for j in range(8):
            rows_a[i, pl.ds(j * 16, 16)] = z16
        return carry
    lax.fori_loop(0, 128, _zb, 0)
    for q in range(5):
        pltpu.sync_copy(rows_a, acc_sh.at[pl.ds(s * RPT + q * 128, 128)])
    plsc.subcore_barrier()

    ebase = s * 10240
    roff = c * NP

    def _load_idx(k, e2, srcv, dstv):
        pltpu.sync_copy(epad_hbm.at[:, pl.ds(ebase + k * 128, 128)], e2)
        for j in range(8):
            sl = pl.ds(j * 16, 16)
            srcv[sl] = e2[0, sl] + roff
            dstv[sl] = e2[1, sl]

    def _half(k, e2, srcv, dstv, rows, gsem, ssem):
        pltpu.make_async_copy(zin_hbm.at[srcv], rows, gsem).wait()
        pltpu.async_copy(rows, acc_sh.at[dstv], ssem, add=True)
        pltpu.make_async_copy(rows, acc_sh.at[dstv], ssem).wait()

        @pl.when(k + 2 < 80)
        def _():
            _load_idx(k + 2, e2, srcv, dstv)
            pltpu.async_copy(zin_hbm.at[srcv], rows, gsem)

    _load_idx(0, e2_a, src_a, dst_a)
    pltpu.async_copy(zin_hbm.at[src_a], rows_a, gsem_a)
    _load_idx(1, e2_b, src_b, dst_b)
    pltpu.async_copy(zin_hbm.at[src_b], rows_b, gsem_b)

    def _body(t, carry):
        _half(2 * t, e2_a, src_a, dst_a, rows_a, gsem_a, ssem_a)
        _half(2 * t + 1, e2_b, src_b, dst_b, rows_b, gsem_b, ssem_b)
        return carry
    lax.fori_loop(0, 40, _body, 0)
    plsc.subcore_barrier()
    pltpu.sync_copy(acc_sh.at[pl.ds(s * RPT, RPT)],
                    zout_hbm.at[pl.ds(c * NP + s * RPT, RPT)])


def _conv_call(eidx, zin):
    return pl.kernel(
        _conv_body,
        out_type=jax.ShapeDtypeStruct((2 * NP, 128), jnp.float32),
        mesh=_MESH,
        compiler_params=_SC_PARAMS,
        scratch_types=[
            pltpu.VMEM_SHARED((NP, 128), jnp.float32),
            pltpu.VMEM((2, 128), jnp.int32),
            pltpu.VMEM((2, 128), jnp.int32),
            pltpu.VMEM((128,), jnp.int32),
            pltpu.VMEM((128,), jnp.int32),
            pltpu.VMEM((128,), jnp.int32),
            pltpu.VMEM((128,), jnp.int32),
            pltpu.VMEM((128, 128), jnp.float32),
            pltpu.VMEM((128, 128), jnp.float32),
            pltpu.SemaphoreType.DMA,
            pltpu.SemaphoreType.DMA,
            pltpu.SemaphoreType.DMA,
            pltpu.SemaphoreType.DMA,
        ],
    )(eidx, zin)


def _scale_body(z_ref, d2_ref, o_ref):
    o_ref[...] = z_ref[...] * d2_ref[...]


def _scale_call(z2, d2p):
    return pl.pallas_call(
        _scale_body,
        grid=(32,),
        in_specs=[
            pl.BlockSpec((RPT, 128), lambda j: (j, 0)),
            pl.BlockSpec((RPT, 1), lambda j: (j % 16, 0)),
        ],
        out_specs=pl.BlockSpec((RPT, 128), lambda j: (j, 0)),
        out_shape=jax.ShapeDtypeStruct((2 * NP, 128), jnp.float32),
    )(z2, d2p)


def _dot_body(ei_hbm, z4_hbm, d_hbm, out_hbm, dv, e2_a, e2_b,
              ai_a, bi_a, ai_b, bi_b, rows_aa, rows_ab, rows_ba, rows_bb,
              outbuf, ga1, ga2, gb1, gb2):
    c = lax.axis_index("c")
    s = lax.axis_index("s")
    ebase = s * 10240
    roff = c * NP

    pltpu.sync_copy(d_hbm, dv)
    lane = lax.iota(jnp.int32, 16)

    def _issue(k, e2, ai, bi, ra, rb, s1, s2):
        pltpu.sync_copy(ei_hbm.at[:, pl.ds(ebase + k * 128, 128)], e2)
        for j in range(8):
            sl = pl.ds(j * 16, 16)
            ai[sl] = e2[0, sl] + roff
            bi[sl] = e2[1, sl] + roff
        pltpu.async_copy(z4_hbm.at[ai], ra, s1)
        pltpu.async_copy(z4_hbm.at[bi], rb, s2)

    def _compute(k, e2, ai, bi, ra, rb, s1, s2):
        pltpu.make_async_copy(z4_hbm.at[ai], ra, s1).wait()
        pltpu.make_async_copy(z4_hbm.at[bi], rb, s2).wait()
        for g in range(8):
            rowidx = g * 16 + lane

            def _cj(jo, accs):
                j0 = jo * 8
                out = []
                for jj in range(8):
                    cj = (jnp.full((16,), jj, jnp.int32) + j0 + lane) & 127
                    av = plsc.load_gather(ra, [rowidx, cj])
                    bv = plsc.load_gather(rb, [rowidx, cj])
                    out.append(accs[jj] + av * bv)
                return tuple(out)
            accs = lax.fori_loop(0, 16, _cj, (_Z16(),) * 8)
            res = (((accs[0] + accs[1]) + (accs[2] + accs[3]))
                   + ((accs[4] + accs[5]) + (accs[6] + accs[7])))
            sl = pl.ds(g * 16, 16)
            da = plsc.load_gather(dv, [e2[0, sl]])
            db = plsc.load_gather(dv, [e2[1, sl]])
            outbuf[sl] = res * da * db
        pltpu.sync_copy(outbuf, out_hbm.at[c, pl.ds(ebase + k * 128, 128)])

    _issue(0, e2_a, ai_a, bi_a, rows_aa, rows_ab, ga1, ga2)
    _issue(1, e2_b, ai_b, bi_b, rows_ba, rows_bb, gb1, gb2)

    def _body(t, carry):
        k = 2 * t
        _compute(k, e2_a, ai_a, bi_a, rows_aa, rows_ab, ga1, ga2)

        @pl.when(k + 2 < 80)
        def _():
            _issue(k + 2, e2_a, ai_a, bi_a, rows_aa, rows_ab, ga1, ga2)
        _compute(k + 1, e2_b, ai_b, bi_b, rows_ba, rows_bb, gb1, gb2)

        @pl.when(k + 3 < 80)
        def _():
            _issue(k + 3, e2_b, ai_b, bi_b, rows_ba, rows_bb, gb1, gb2)
        return carry
    lax.fori_loop(0, 40, _body, 0)


_dot_call = functools.partial(
    pl.kernel,
    out_type=jax.ShapeDtypeStruct((NC, EPAD), jnp.float32),
    mesh=_MESH,
    compiler_params=_SC_PARAMS,
    scratch_types=[
        pltpu.VMEM((NP,), jnp.float32),
        pltpu.VMEM((2, 128), jnp.int32),
        pltpu.VMEM((2, 128), jnp.int32),
        pltpu.VMEM((128,), jnp.int32),
        pltpu.VMEM((128,), jnp.int32),
        pltpu.VMEM((128,), jnp.int32),
        pltpu.VMEM((128,), jnp.int32),
        pltpu.VMEM((128, 128), jnp.float32),
        pltpu.VMEM((128, 128), jnp.float32),
        pltpu.VMEM((128, 128), jnp.float32),
        pltpu.VMEM((128, 128), jnp.float32),
        pltpu.VMEM((128,), jnp.float32),
        pltpu.SemaphoreType.DMA,
        pltpu.SemaphoreType.DMA,
        pltpu.SemaphoreType.DMA,
        pltpu.SemaphoreType.DMA,
    ],
)(_dot_body)


def _comb_body(p_ref, o_ref):
    o_ref[...] = p_ref[0, :] + p_ref[1, :]


def _comb_call(partials):
    return pl.pallas_call(
        _comb_body,
        grid=(80,),
        in_specs=[pl.BlockSpec((2, 2048), lambda j: (0, j))],
        out_specs=pl.BlockSpec((2048,), lambda j: (j,)),
        out_shape=jax.ShapeDtypeStruct((EPAD,), jnp.float32),
    )(partials)


def kernel(x_input, edge_index_input, pos_edge_index, W, b):
    padv = (N + jnp.arange(EPAD - E, dtype=jnp.int32) % (NP - N))
    epad = jnp.concatenate(
        [pos_edge_index, jnp.stack([padv, padv])], axis=1)
    eipad = jnp.concatenate(
        [edge_index_input, jnp.stack([padv, padv])], axis=1)
    xp = jnp.pad(x_input, ((0, NP - N), (0, 0)))
    b2 = b.reshape(2, 1, 128)

    degp = _deg_call(epad)
    x1h3, dcol, d2col = _linear_call(xp, W, b2, degp.reshape(2, NP, 1))
    x1h = x1h3.reshape(2 * NP, 128)

    z2 = _conv_call(epad, x1h)
    z3 = _scale_call(z2, d2col)
    z4 = _conv_call(epad, z3)
    partials = _dot_call(eipad, z4, dcol.reshape(NP))
    logits = _comb_call(partials)
    return logits[:E]


# speedup vs baseline: 3.0775x; 1.0106x over previous
import functools

import jax
import jax.numpy as jnp
from jax import lax
from jax.experimental import pallas as pl
from jax.experimental.pallas import tpu as pltpu
from jax.experimental.pallas import tpu_sc as plsc

N = 10000
D = 256
E = 160000
NC = 2
NS = 16
NP = 10240
RPT = 640
EPAD = 163840

_MESH = plsc.VectorSubcoreMesh(core_axis_name="c", subcore_axis_name="s",
                               num_cores=NC, num_subcores=NS)
_SC_PARAMS = pltpu.CompilerParams(needs_layout_passes=False)

_Z16 = functools.partial(jnp.zeros, (16,), jnp.float32)


def _deg_body(epad_hbm, deg_hbm, acc_sh, zbuf, ones_v, idx_a, idx_b,
              sem_a, sem_b):
    c = lax.axis_index("c")
    s = lax.axis_index("s")
    wid = c * NS + s

    z16 = _Z16()
    o16 = jnp.ones((16,), jnp.float32)

    def _zb(i, carry):
        zbuf[pl.ds(i * 16, 16)] = z16
        return carry
    lax.fori_loop(0, RPT // 16, _zb, 0)
    for j in range(8):
        ones_v[pl.ds(j * 16, 16)] = o16
    pltpu.sync_copy(zbuf, acc_sh.at[pl.ds(s * RPT, RPT)])
    plsc.subcore_barrier()

    ebase = wid * 5120

    def _issue(k, idx, sem):
        pltpu.sync_copy(epad_hbm.at[1, pl.ds(ebase + k * 128, 128)], idx)
        pltpu.async_copy(ones_v, acc_sh.at[idx], sem, add=True)

    _issue(0, idx_a, sem_a)
    _issue(1, idx_b, sem_b)

    def _body(t, carry):
        k = 2 * t
        pltpu.make_async_copy(ones_v, acc_sh.at[idx_a], sem_a).wait()

        @pl.when(k + 2 < 40)
        def _():
            _issue(k + 2, idx_a, sem_a)
        pltpu.make_async_copy(ones_v, acc_sh.at[idx_b], sem_b).wait()

        @pl.when(k + 3 < 40)
        def _():
            _issue(k + 3, idx_b, sem_b)
        return carry
    lax.fori_loop(0, 20, _body, 0)
    plsc.subcore_barrier()
    pltpu.sync_copy(acc_sh.at[pl.ds(s * RPT, RPT)],
                    deg_hbm.at[c, pl.ds(s * RPT, RPT)])


_deg_call = functools.partial(
    pl.kernel,
    out_type=jax.ShapeDtypeStruct((NC, NP), jnp.float32),
    mesh=_MESH,
    compiler_params=_SC_PARAMS,
    scratch_types=[
        pltpu.VMEM_SHARED((NP,), jnp.float32),
        pltpu.VMEM((RPT,), jnp.float32),
        pltpu.VMEM((128,), jnp.float32),
        pltpu.VMEM((128,), jnp.int32),
        pltpu.VMEM((128,), jnp.int32),
        pltpu.SemaphoreType.DMA,
        pltpu.SemaphoreType.DMA,
    ],
)(_deg_body)


def _linear_body(x_ref, w_ref, b_ref, degp_ref, xh_ref, d_ref, d2_ref):
    deg = degp_ref[0, :, 0] + degp_ref[1, :, 0]
    dv = jnp.where(deg > 0, lax.rsqrt(deg), 0.0)
    res = lax.dot_general(x_ref[...], w_ref[...], (((1,), (1,)), ((), ())),
                          preferred_element_type=jnp.float32)
    xh_ref[0] = (res + b_ref[0]) * dv[:, None]
    d_ref[...] = dv[:, None]
    d2_ref[...] = (dv * dv)[:, None]


def _linear_call(x, w, b2, degp):
    return pl.pallas_call(
        _linear_body,
        grid=(16, 2),
        in_specs=[
            pl.BlockSpec((RPT, 256), lambda i, h: (i, 0)),
            pl.BlockSpec((128, 256), lambda i, h: (h, 0)),
            pl.BlockSpec((1, 1, 128), lambda i, h: (h, 0, 0)),
            pl.BlockSpec((2, RPT, 1), lambda i, h: (0, i, 0)),
        ],
        out_specs=[
            pl.BlockSpec((1, RPT, 128), lambda i, h: (h, i, 0)),
            pl.BlockSpec((RPT, 1), lambda i, h: (i, 0)),
            pl.BlockSpec((RPT, 1), lambda i, h: (i, 0)),
        ],
        out_shape=[
            jax.ShapeDtypeStruct((2, NP, 128), jnp.float32),
            jax.ShapeDtypeStruct((NP, 1), jnp.float32),
            jax.ShapeDtypeStruct((NP, 1), jnp.float32),
        ],
    )(x, w, b2, degp)


def _conv_body(epad_hbm, zin_hbm, zout_hbm, acc_sh,
               e2_a, e2_b, src_a, dst_a, src_b, dst_b, rows_a, rows_b,
               gsem_a, gsem_b, ssem_a, ssem_b):
    c = lax.axis_index("c")
    s = lax.axis_index("s")

    z16 = _Z16()

    def _zb(i, carry):
        for j in range(8):
            rows_a[i, pl.ds(j * 16, 16)] = z16
        return carry
    lax.fori_loop(0, 128, _zb, 0)
    for q in range(5):
        pltpu.sync_copy(rows_a, acc_sh.at[pl.ds(s * RPT + q * 128, 128)])
    plsc.subcore_barrier()

    ebase = s * 10240
    roff = c * NP

    def _load_idx(k, e2, srcv, dstv):
        pltpu.sync_copy(epad_hbm.at[:, pl.ds(ebase + k * 128, 128)], e2)
        for j in range(8):
            sl = pl.ds(j * 16, 16)
            srcv[sl] = e2[0, sl] + roff
            dstv[sl] = e2[1, sl]

    def _half(k, e2, srcv, dstv, rows, gsem, ssem):
        pltpu.make_async_copy(zin_hbm.at[srcv], rows, gsem).wait()
        pltpu.async_copy(rows, acc_sh.at[dstv], ssem, add=True)
        pltpu.make_async_copy(rows, acc_sh.at[dstv], ssem).wait()

        @pl.when(k + 2 < 80)
        def _():
            _load_idx(k + 2, e2, srcv, dstv)
            pltpu.async_copy(zin_hbm.at[srcv], rows, gsem)

    _load_idx(0, e2_a, src_a, dst_a)
    pltpu.async_copy(zin_hbm.at[src_a], rows_a, gsem_a)
    _load_idx(1, e2_b, src_b, dst_b)
    pltpu.async_copy(zin_hbm.at[src_b], rows_b, gsem_b)

    def _body(t, carry):
        _half(2 * t, e2_a, src_a, dst_a, rows_a, gsem_a, ssem_a)
        _half(2 * t + 1, e2_b, src_b, dst_b, rows_b, gsem_b, ssem_b)
        return carry
    lax.fori_loop(0, 40, _body, 0)
    plsc.subcore_barrier()
    pltpu.sync_copy(acc_sh.at[pl.ds(s * RPT, RPT)],
                    zout_hbm.at[pl.ds(c * NP + s * RPT, RPT)])


def _conv_call(eidx, zin):
    return pl.kernel(
        _conv_body,
        out_type=jax.ShapeDtypeStruct((2 * NP, 128), jnp.float32),
        mesh=_MESH,
        compiler_params=_SC_PARAMS,
        scratch_types=[
            pltpu.VMEM_SHARED((NP, 128), jnp.float32),
            pltpu.VMEM((2, 128), jnp.int32),
            pltpu.VMEM((2, 128), jnp.int32),
            pltpu.VMEM((128,), jnp.int32),
            pltpu.VMEM((128,), jnp.int32),
            pltpu.VMEM((128,), jnp.int32),
            pltpu.VMEM((128,), jnp.int32),
            pltpu.VMEM((128, 128), jnp.float32),
            pltpu.VMEM((128, 128), jnp.float32),
            pltpu.SemaphoreType.DMA,
            pltpu.SemaphoreType.DMA,
            pltpu.SemaphoreType.DMA,
            pltpu.SemaphoreType.DMA,
        ],
    )(eidx, zin)


def _scale_body(z_ref, d2_ref, o_ref):
    o_ref[...] = z_ref[...] * d2_ref[...]


def _scale_call(z2, d2p):
    return pl.pallas_call(
        _scale_body,
        grid=(32,),
        in_specs=[
            pl.BlockSpec((RPT, 128), lambda j: (j, 0)),
            pl.BlockSpec((RPT, 1), lambda j: (j % 16, 0)),
        ],
        out_specs=pl.BlockSpec((RPT, 128), lambda j: (j, 0)),
        out_shape=jax.ShapeDtypeStruct((2 * NP, 128), jnp.float32),
    )(z2, d2p)


def _dot_body(ei_hbm, z4_hbm, d_hbm, out_hbm, dv, e2_a, e2_b, e2_c,
              ai_a, bi_a, ai_b, bi_b, ai_c, bi_c, rows_aa, rows_ab,
              rows_ba, rows_bb, rows_ca, rows_cb,
              outbuf, ga1, ga2, gb1, gb2, gc1, gc2):
    c = lax.axis_index("c")
    s = lax.axis_index("s")
    ebase = s * 10240
    roff = c * NP

    pltpu.sync_copy(d_hbm, dv)
    lane = lax.iota(jnp.int32, 16)

    def _issue(k, e2, ai, bi, ra, rb, s1, s2):
        pltpu.sync_copy(ei_hbm.at[:, pl.ds(ebase + k * 128, 128)], e2)
        for j in range(8):
            sl = pl.ds(j * 16, 16)
            ai[sl] = e2[0, sl] + roff
            bi[sl] = e2[1, sl] + roff
        pltpu.async_copy(z4_hbm.at[ai], ra, s1)
        pltpu.async_copy(z4_hbm.at[bi], rb, s2)

    def _compute(k, e2, ai, bi, ra, rb, s1, s2):
        pltpu.make_async_copy(z4_hbm.at[ai], ra, s1).wait()
        pltpu.make_async_copy(z4_hbm.at[bi], rb, s2).wait()
        for g in range(8):
            rowidx = g * 16 + lane

            def _cj(jo, accs):
                j0 = jo * 8
                out = []
                for jj in range(8):
                    cj = (jnp.full((16,), jj, jnp.int32) + j0 + lane) & 127
                    av = plsc.load_gather(ra, [rowidx, cj])
                    bv = plsc.load_gather(rb, [rowidx, cj])
                    out.append(accs[jj] + av * bv)
                return tuple(out)
            accs = lax.fori_loop(0, 16, _cj, (_Z16(),) * 8)
            res = (((accs[0] + accs[1]) + (accs[2] + accs[3]))
                   + ((accs[4] + accs[5]) + (accs[6] + accs[7])))
            sl = pl.ds(g * 16, 16)
            da = plsc.load_gather(dv, [e2[0, sl]])
            db = plsc.load_gather(dv, [e2[1, sl]])
            outbuf[sl] = res * da * db
        pltpu.sync_copy(outbuf, out_hbm.at[c, pl.ds(ebase + k * 128, 128)])

    sets = ((e2_a, ai_a, bi_a, rows_aa, rows_ab, ga1, ga2),
            (e2_b, ai_b, bi_b, rows_ba, rows_bb, gb1, gb2),
            (e2_c, ai_c, bi_c, rows_ca, rows_cb, gc1, gc2))
    _issue(0, *sets[0])
    _issue(1, *sets[1])
    _issue(2, *sets[2])

    def _body(t, carry):
        k = 3 * t
        for q in range(3):
            _compute(k + q, *sets[q])

            @pl.when(k + q + 3 < 80)
            def _():
                _issue(k + q + 3, *sets[q])
        return carry
    lax.fori_loop(0, 26, _body, 0)
    _compute(78, *sets[0])
    _compute(79, *sets[1])


_dot_call = functools.partial(
    pl.kernel,
    out_type=jax.ShapeDtypeStruct((NC, EPAD), jnp.float32),
    mesh=_MESH,
    compiler_params=_SC_PARAMS,
    scratch_types=[
        pltpu.VMEM((NP,), jnp.float32),
        pltpu.VMEM((2, 128), jnp.int32),
        pltpu.VMEM((2, 128), jnp.int32),
        pltpu.VMEM((2, 128), jnp.int32),
        pltpu.VMEM((128,), jnp.int32),
        pltpu.VMEM((128,), jnp.int32),
        pltpu.VMEM((128,), jnp.int32),
        pltpu.VMEM((128,), jnp.int32),
        pltpu.VMEM((128,), jnp.int32),
        pltpu.VMEM((128,), jnp.int32),
        pltpu.VMEM((128, 128), jnp.float32),
        pltpu.VMEM((128, 128), jnp.float32),
        pltpu.VMEM((128, 128), jnp.float32),
        pltpu.VMEM((128, 128), jnp.float32),
        pltpu.VMEM((128, 128), jnp.float32),
        pltpu.VMEM((128, 128), jnp.float32),
        pltpu.VMEM((128,), jnp.float32),
        pltpu.SemaphoreType.DMA,
        pltpu.SemaphoreType.DMA,
        pltpu.SemaphoreType.DMA,
        pltpu.SemaphoreType.DMA,
        pltpu.SemaphoreType.DMA,
        pltpu.SemaphoreType.DMA,
    ],
)(_dot_body)


def _comb_body(p_ref, o_ref):
    o_ref[...] = p_ref[0, :] + p_ref[1, :]


def _comb_call(partials):
    return pl.pallas_call(
        _comb_body,
        grid=(80,),
        in_specs=[pl.BlockSpec((2, 2048), lambda j: (0, j))],
        out_specs=pl.BlockSpec((2048,), lambda j: (j,)),
        out_shape=jax.ShapeDtypeStruct((EPAD,), jnp.float32),
    )(partials)


def kernel(x_input, edge_index_input, pos_edge_index, W, b):
    padv = (N + jnp.arange(EPAD - E, dtype=jnp.int32) % (NP - N))
    epad = jnp.concatenate(
        [pos_edge_index, jnp.stack([padv, padv])], axis=1)
    eipad = jnp.concatenate(
        [edge_index_input, jnp.stack([padv, padv])], axis=1)
    b2 = b.reshape(2, 1, 128)

    degp = _deg_call(epad)
    x1h3, dcol, d2col = _linear_call(x_input, W, b2, degp.reshape(2, NP, 1))
    x1h = x1h3.reshape(2 * NP, 128)

    z2 = _conv_call(epad, x1h)
    z3 = _scale_call(z2, d2col)
    z4 = _conv_call(epad, z3)
    partials = _dot_call(eipad, z4, dcol.reshape(NP))
    logits = _comb_call(partials)
    return logits[:E]


# speedup vs baseline: 3.1137x; 1.0118x over previous
import functools

import jax
import jax.numpy as jnp
from jax import lax
from jax.experimental import pallas as pl
from jax.experimental.pallas import tpu as pltpu
from jax.experimental.pallas import tpu_sc as plsc

N = 10000
D = 256
E = 160000
NC = 2
NS = 16
NP = 10240
RPT = 640
EPAD = 163840

_MESH = plsc.VectorSubcoreMesh(core_axis_name="c", subcore_axis_name="s",
                               num_cores=NC, num_subcores=NS)
_SC_PARAMS = pltpu.CompilerParams(needs_layout_passes=False)

_Z16 = functools.partial(jnp.zeros, (16,), jnp.float32)


def _deg_body(epad_hbm, deg_hbm, acc_sh, zbuf, ones_v, idx_a, idx_b,
              sem_a, sem_b):
    c = lax.axis_index("c")
    s = lax.axis_index("s")
    wid = c * NS + s

    z16 = _Z16()
    o16 = jnp.ones((16,), jnp.float32)

    def _zb(i, carry):
        zbuf[pl.ds(i * 16, 16)] = z16
        return carry
    lax.fori_loop(0, RPT // 16, _zb, 0)
    for j in range(8):
        ones_v[pl.ds(j * 16, 16)] = o16
    pltpu.sync_copy(zbuf, acc_sh.at[pl.ds(s * RPT, RPT)])
    plsc.subcore_barrier()

    ebase = wid * 5120

    def _issue(k, idx, sem):
        pltpu.sync_copy(epad_hbm.at[1, pl.ds(ebase + k * 128, 128)], idx)
        pltpu.async_copy(ones_v, acc_sh.at[idx], sem, add=True)

    _issue(0, idx_a, sem_a)
    _issue(1, idx_b, sem_b)

    def _body(t, carry):
        k = 2 * t
        pltpu.make_async_copy(ones_v, acc_sh.at[idx_a], sem_a).wait()

        @pl.when(k + 2 < 40)
        def _():
            _issue(k + 2, idx_a, sem_a)
        pltpu.make_async_copy(ones_v, acc_sh.at[idx_b], sem_b).wait()

        @pl.when(k + 3 < 40)
        def _():
            _issue(k + 3, idx_b, sem_b)
        return carry
    lax.fori_loop(0, 20, _body, 0)
    plsc.subcore_barrier()
    pltpu.sync_copy(acc_sh.at[pl.ds(s * RPT, RPT)],
                    deg_hbm.at[c, pl.ds(s * RPT, RPT)])


_deg_call = functools.partial(
    pl.kernel,
    out_type=jax.ShapeDtypeStruct((NC, NP), jnp.float32),
    mesh=_MESH,
    compiler_params=_SC_PARAMS,
    scratch_types=[
        pltpu.VMEM_SHARED((NP,), jnp.float32),
        pltpu.VMEM((RPT,), jnp.float32),
        pltpu.VMEM((128,), jnp.float32),
        pltpu.VMEM((128,), jnp.int32),
        pltpu.VMEM((128,), jnp.int32),
        pltpu.SemaphoreType.DMA,
        pltpu.SemaphoreType.DMA,
    ],
)(_deg_body)


def _linear_body(x_ref, w_ref, b_ref, degp_ref, xh_ref, d_ref, d2_ref):
    deg = degp_ref[0, :, 0] + degp_ref[1, :, 0]
    dv = jnp.where(deg > 0, lax.rsqrt(deg), 0.0)
    res = lax.dot_general(x_ref[...], w_ref[...], (((1,), (1,)), ((), ())),
                          preferred_element_type=jnp.float32)
    xh_ref[0] = (res + b_ref[0]) * dv[:, None]
    d_ref[...] = dv[:, None]
    d2_ref[...] = (dv * dv)[:, None]


def _linear_call(x, w, b2, degp):
    return pl.pallas_call(
        _linear_body,
        grid=(16, 2),
        in_specs=[
            pl.BlockSpec((RPT, 256), lambda i, h: (i, 0)),
            pl.BlockSpec((128, 256), lambda i, h: (h, 0)),
            pl.BlockSpec((1, 1, 128), lambda i, h: (h, 0, 0)),
            pl.BlockSpec((2, RPT, 1), lambda i, h: (0, i, 0)),
        ],
        out_specs=[
            pl.BlockSpec((1, RPT, 128), lambda i, h: (h, i, 0)),
            pl.BlockSpec((RPT, 1), lambda i, h: (i, 0)),
            pl.BlockSpec((RPT, 1), lambda i, h: (i, 0)),
        ],
        out_shape=[
            jax.ShapeDtypeStruct((2, NP, 128), jnp.float32),
            jax.ShapeDtypeStruct((NP, 1), jnp.float32),
            jax.ShapeDtypeStruct((NP, 1), jnp.float32),
        ],
    )(x, w, b2, degp)


def _conv_body(epad_hbm, zin_hbm, zout_hbm, acc_sh,
               src_a0, dst_a0, src_a1, dst_a1, src_b0, dst_b0, src_b1,
               dst_b1, rows_a, rows_b, gsem_a, gsem_b, ssem_a, ssem_b):
    c = lax.axis_index("c")
    s = lax.axis_index("s")

    z16 = _Z16()

    def _zb(i, carry):
        for j in range(8):
            rows_a[i, pl.ds(j * 16, 16)] = z16
        return carry
    lax.fori_loop(0, 128, _zb, 0)
    for q in range(5):
        pltpu.sync_copy(rows_a, acc_sh.at[pl.ds(s * RPT + q * 128, 128)])
    plsc.subcore_barrier()

    ebase = s * 10240
    roff = c * NP

    def _load_idx(k, srcv, dstv):
        pltpu.sync_copy(epad_hbm.at[0, pl.ds(ebase + k * 128, 128)], srcv)
        pltpu.sync_copy(epad_hbm.at[1, pl.ds(ebase + k * 128, 128)], dstv)
        for j in range(8):
            sl = pl.ds(j * 16, 16)
            srcv[sl] = srcv[sl] + roff

    def _half(k, srcv, dstv, srcv2, dstv2, rows, gsem, ssem):
        pltpu.make_async_copy(zin_hbm.at[srcv], rows, gsem).wait()
        pltpu.async_copy(rows, acc_sh.at[dstv], ssem, add=True)

        @pl.when(k + 2 < 80)
        def _():
            _load_idx(k + 2, srcv2, dstv2)
        pltpu.make_async_copy(rows, acc_sh.at[dstv], ssem).wait()

        @pl.when(k + 2 < 80)
        def _():
            pltpu.async_copy(zin_hbm.at[srcv2], rows, gsem)

    _load_idx(0, src_a0, dst_a0)
    pltpu.async_copy(zin_hbm.at[src_a0], rows_a, gsem_a)
    _load_idx(1, src_b0, dst_b0)
    pltpu.async_copy(zin_hbm.at[src_b0], rows_b, gsem_b)

    def _body(t, carry):
        k = 4 * t
        _half(k, src_a0, dst_a0, src_a1, dst_a1, rows_a, gsem_a, ssem_a)
        _half(k + 1, src_b0, dst_b0, src_b1, dst_b1, rows_b, gsem_b, ssem_b)
        _half(k + 2, src_a1, dst_a1, src_a0, dst_a0, rows_a, gsem_a, ssem_a)
        _half(k + 3, src_b1, dst_b1, src_b0, dst_b0, rows_b, gsem_b, ssem_b)
        return carry
    lax.fori_loop(0, 20, _body, 0)
    plsc.subcore_barrier()
    pltpu.sync_copy(acc_sh.at[pl.ds(s * RPT, RPT)],
                    zout_hbm.at[pl.ds(c * NP + s * RPT, RPT)])


def _conv_call(eidx, zin):
    return pl.kernel(
        _conv_body,
        out_type=jax.ShapeDtypeStruct((2 * NP, 128), jnp.float32),
        mesh=_MESH,
        compiler_params=_SC_PARAMS,
        scratch_types=[
            pltpu.VMEM_SHARED((NP, 128), jnp.float32),
            pltpu.VMEM((128,), jnp.int32),
            pltpu.VMEM((128,), jnp.int32),
            pltpu.VMEM((128,), jnp.int32),
            pltpu.VMEM((128,), jnp.int32),
            pltpu.VMEM((128,), jnp.int32),
            pltpu.VMEM((128,), jnp.int32),
            pltpu.VMEM((128,), jnp.int32),
            pltpu.VMEM((128,), jnp.int32),
            pltpu.VMEM((128, 128), jnp.float32),
            pltpu.VMEM((128, 128), jnp.float32),
            pltpu.SemaphoreType.DMA,
            pltpu.SemaphoreType.DMA,
            pltpu.SemaphoreType.DMA,
            pltpu.SemaphoreType.DMA,
        ],
    )(eidx, zin)


def _scale_body(z_ref, d2_ref, o_ref):
    o_ref[...] = z_ref[...] * d2_ref[...]


def _scale_call(z2, d2p):
    return pl.pallas_call(
        _scale_body,
        grid=(32,),
        in_specs=[
            pl.BlockSpec((RPT, 128), lambda j: (j, 0)),
            pl.BlockSpec((RPT, 1), lambda j: (j % 16, 0)),
        ],
        out_specs=pl.BlockSpec((RPT, 128), lambda j: (j, 0)),
        out_shape=jax.ShapeDtypeStruct((2 * NP, 128), jnp.float32),
    )(z2, d2p)


def _dot_body(ei_hbm, z4_hbm, d_hbm, out_hbm, dv, e2_a, e2_b, e2_c,
              ai_a, bi_a, ai_b, bi_b, ai_c, bi_c, rows_aa, rows_ab,
              rows_ba, rows_bb, rows_ca, rows_cb,
              outbuf, ga1, ga2, gb1, gb2, gc1, gc2):
    c = lax.axis_index("c")
    s = lax.axis_index("s")
    ebase = s * 10240
    roff = c * NP

    pltpu.sync_copy(d_hbm, dv)
    lane = lax.iota(jnp.int32, 16)

    def _issue(k, e2, ai, bi, ra, rb, s1, s2):
        pltpu.sync_copy(ei_hbm.at[:, pl.ds(ebase + k * 128, 128)], e2)
        for j in range(8):
            sl = pl.ds(j * 16, 16)
            ai[sl] = e2[0, sl] + roff
            bi[sl] = e2[1, sl] + roff
        pltpu.async_copy(z4_hbm.at[ai], ra, s1)
        pltpu.async_copy(z4_hbm.at[bi], rb, s2)

    def _compute(k, e2, ai, bi, ra, rb, s1, s2):
        pltpu.make_async_copy(z4_hbm.at[ai], ra, s1).wait()
        pltpu.make_async_copy(z4_hbm.at[bi], rb, s2).wait()
        for g in range(8):
            rowidx = g * 16 + lane

            def _cj(jo, accs):
                j0 = jo * 8
                out = []
                for jj in range(8):
                    cj = (jnp.full((16,), jj, jnp.int32) + j0 + lane) & 127
                    av = plsc.load_gather(ra, [rowidx, cj])
                    bv = plsc.load_gather(rb, [rowidx, cj])
                    out.append(accs[jj] + av * bv)
                return tuple(out)
            accs = lax.fori_loop(0, 16, _cj, (_Z16(),) * 8)
            res = (((accs[0] + accs[1]) + (accs[2] + accs[3]))
                   + ((accs[4] + accs[5]) + (accs[6] + accs[7])))
            sl = pl.ds(g * 16, 16)
            da = plsc.load_gather(dv, [e2[0, sl]])
            db = plsc.load_gather(dv, [e2[1, sl]])
            outbuf[sl] = res * da * db
        pltpu.sync_copy(outbuf, out_hbm.at[c, pl.ds(ebase + k * 128, 128)])

    sets = ((e2_a, ai_a, bi_a, rows_aa, rows_ab, ga1, ga2),
            (e2_b, ai_b, bi_b, rows_ba, rows_bb, gb1, gb2),
            (e2_c, ai_c, bi_c, rows_ca, rows_cb, gc1, gc2))
    _issue(0, *sets[0])
    _issue(1, *sets[1])
    _issue(2, *sets[2])

    def _body(t, carry):
        k = 3 * t
        for q in range(3):
            _compute(k + q, *sets[q])

            @pl.when(k + q + 3 < 80)
            def _():
                _issue(k + q + 3, *sets[q])
        return carry
    lax.fori_loop(0, 26, _body, 0)
    _compute(78, *sets[0])
    _compute(79, *sets[1])


_dot_call = functools.partial(
    pl.kernel,
    out_type=jax.ShapeDtypeStruct((NC, EPAD), jnp.float32),
    mesh=_MESH,
    compiler_params=_SC_PARAMS,
    scratch_types=[
        pltpu.VMEM((NP,), jnp.float32),
        pltpu.VMEM((2, 128), jnp.int32),
        pltpu.VMEM((2, 128), jnp.int32),
        pltpu.VMEM((2, 128), jnp.int32),
        pltpu.VMEM((128,), jnp.int32),
        pltpu.VMEM((128,), jnp.int32),
        pltpu.VMEM((128,), jnp.int32),
        pltpu.VMEM((128,), jnp.int32),
        pltpu.VMEM((128,), jnp.int32),
        pltpu.VMEM((128,), jnp.int32),
        pltpu.VMEM((128, 128), jnp.float32),
        pltpu.VMEM((128, 128), jnp.float32),
        pltpu.VMEM((128, 128), jnp.float32),
        pltpu.VMEM((128, 128), jnp.float32),
        pltpu.VMEM((128, 128), jnp.float32),
        pltpu.VMEM((128, 128), jnp.float32),
        pltpu.VMEM((128,), jnp.float32),
        pltpu.SemaphoreType.DMA,
        pltpu.SemaphoreType.DMA,
        pltpu.SemaphoreType.DMA,
        pltpu.SemaphoreType.DMA,
        pltpu.SemaphoreType.DMA,
        pltpu.SemaphoreType.DMA,
    ],
)(_dot_body)


def _comb_body(p_ref, o_ref):
    o_ref[...] = p_ref[0, :] + p_ref[1, :]


def _comb_call(partials):
    return pl.pallas_call(
        _comb_body,
        grid=(80,),
        in_specs=[pl.BlockSpec((2, 2048), lambda j: (0, j))],
        out_specs=pl.BlockSpec((2048,), lambda j: (j,)),
        out_shape=jax.ShapeDtypeStruct((EPAD,), jnp.float32),
    )(partials)


def kernel(x_input, edge_index_input, pos_edge_index, W, b):
    padv = (N + jnp.arange(EPAD - E, dtype=jnp.int32) % (NP - N))
    epad = jnp.concatenate(
        [pos_edge_index, jnp.stack([padv, padv])], axis=1)
    eipad = jnp.concatenate(
        [edge_index_input, jnp.stack([padv, padv])], axis=1)
    b2 = b.reshape(2, 1, 128)

    degp = _deg_call(epad)
    x1h3, dcol, d2col = _linear_call(x_input, W, b2, degp.reshape(2, NP, 1))
    x1h = x1h3.reshape(2 * NP, 128)

    z2 = _conv_call(epad, x1h)
    z3 = _scale_call(z2, d2col)
    z4 = _conv_call(epad, z3)
    partials = _dot_call(eipad, z4, dcol.reshape(NP))
    logits = _comb_call(partials)
    return logits[:E]


# speedup vs baseline: 3.1300x; 1.0052x over previous
import functools

import jax
import jax.numpy as jnp
from jax import lax
from jax.experimental import pallas as pl
from jax.experimental.pallas import tpu as pltpu
from jax.experimental.pallas import tpu_sc as plsc

N = 10000
D = 256
E = 160000
NC = 2
NS = 16
NP = 10240
RPT = 640
EPAD = 163840

_MESH = plsc.VectorSubcoreMesh(core_axis_name="c", subcore_axis_name="s",
                               num_cores=NC, num_subcores=NS)
_SC_PARAMS = pltpu.CompilerParams(needs_layout_passes=False)

_Z16 = functools.partial(jnp.zeros, (16,), jnp.float32)


def _deg_body(epad_hbm, deg_hbm, acc_sh, zbuf, ones_v, idx_a, idx_b,
              sem_a, sem_b):
    c = lax.axis_index("c")
    s = lax.axis_index("s")
    wid = c * NS + s

    z16 = _Z16()
    o16 = jnp.ones((16,), jnp.float32)

    def _zb(i, carry):
        zbuf[pl.ds(i * 16, 16)] = z16
        return carry
    lax.fori_loop(0, RPT // 16, _zb, 0)
    for j in range(8):
        ones_v[pl.ds(j * 16, 16)] = o16
    pltpu.sync_copy(zbuf, acc_sh.at[pl.ds(s * RPT, RPT)])
    plsc.subcore_barrier()

    ebase = wid * 5120

    def _issue(k, idx, sem):
        pltpu.sync_copy(epad_hbm.at[1, pl.ds(ebase + k * 128, 128)], idx)
        pltpu.async_copy(ones_v, acc_sh.at[idx], sem, add=True)

    _issue(0, idx_a, sem_a)
    _issue(1, idx_b, sem_b)

    def _body(t, carry):
        k = 2 * t
        pltpu.make_async_copy(ones_v, acc_sh.at[idx_a], sem_a).wait()

        @pl.when(k + 2 < 40)
        def _():
            _issue(k + 2, idx_a, sem_a)
        pltpu.make_async_copy(ones_v, acc_sh.at[idx_b], sem_b).wait()

        @pl.when(k + 3 < 40)
        def _():
            _issue(k + 3, idx_b, sem_b)
        return carry
    lax.fori_loop(0, 20, _body, 0)
    plsc.subcore_barrier()
    pltpu.sync_copy(acc_sh.at[pl.ds(s * RPT, RPT)],
                    deg_hbm.at[c, pl.ds(s * RPT, RPT)])


_deg_call = functools.partial(
    pl.kernel,
    out_type=jax.ShapeDtypeStruct((NC, NP), jnp.float32),
    mesh=_MESH,
    compiler_params=_SC_PARAMS,
    scratch_types=[
        pltpu.VMEM_SHARED((NP,), jnp.float32),
        pltpu.VMEM((RPT,), jnp.float32),
        pltpu.VMEM((128,), jnp.float32),
        pltpu.VMEM((128,), jnp.int32),
        pltpu.VMEM((128,), jnp.int32),
        pltpu.SemaphoreType.DMA,
        pltpu.SemaphoreType.DMA,
    ],
)(_deg_body)


def _linear_body(x_ref, w_ref, b_ref, degp_ref, xh_ref, d_ref, d2_ref):
    deg = degp_ref[0, :, 0] + degp_ref[1, :, 0]
    dv = jnp.where(deg > 0, lax.rsqrt(deg), 0.0)
    res = lax.dot_general(x_ref[...], w_ref[...], (((1,), (1,)), ((), ())),
                          preferred_element_type=jnp.float32)
    xh_ref[0] = (res + b_ref[0]) * dv[:, None]
    d_ref[...] = dv[:, None]
    d2_ref[...] = (dv * dv)[:, None]


def _linear_call(x, w, b2, degp):
    return pl.pallas_call(
        _linear_body,
        grid=(16, 2),
        in_specs=[
            pl.BlockSpec((RPT, 256), lambda i, h: (i, 0)),
            pl.BlockSpec((128, 256), lambda i, h: (h, 0)),
            pl.BlockSpec((1, 1, 128), lambda i, h: (h, 0, 0)),
            pl.BlockSpec((2, RPT, 1), lambda i, h: (0, i, 0)),
        ],
        out_specs=[
            pl.BlockSpec((1, RPT, 128), lambda i, h: (h, i, 0)),
            pl.BlockSpec((RPT, 1), lambda i, h: (i, 0)),
            pl.BlockSpec((RPT, 1), lambda i, h: (i, 0)),
        ],
        out_shape=[
            jax.ShapeDtypeStruct((2, NP, 128), jnp.float32),
            jax.ShapeDtypeStruct((NP, 1), jnp.float32),
            jax.ShapeDtypeStruct((NP, 1), jnp.float32),
        ],
    )(x, w, b2, degp)


def _conv_body(epad_hbm, zin_hbm, zout_hbm, acc_sh,
               src_a0, dst_a0, src_a1, dst_a1, src_b0, dst_b0, src_b1,
               dst_b1, rows_a, rows_b, gsem_a, gsem_b, ssem_a, ssem_b):
    c = lax.axis_index("c")
    s = lax.axis_index("s")

    z16 = _Z16()

    def _zb(i, carry):
        for j in range(8):
            rows_a[i, pl.ds(j * 16, 16)] = z16
        return carry
    lax.fori_loop(0, 128, _zb, 0)
    for q in range(5):
        pltpu.sync_copy(rows_a, acc_sh.at[pl.ds(s * RPT + q * 128, 128)])
    plsc.subcore_barrier()

    ebase = s * 10240
    roff = c * NP

    def _load_idx(k, srcv, dstv):
        pltpu.sync_copy(epad_hbm.at[0, pl.ds(ebase + k * 128, 128)], srcv)
        pltpu.sync_copy(epad_hbm.at[1, pl.ds(ebase + k * 128, 128)], dstv)
        for j in range(8):
            sl = pl.ds(j * 16, 16)
            srcv[sl] = srcv[sl] + roff

    def _half(k, srcv, dstv, srcv2, dstv2, rows, gsem, ssem):
        pltpu.make_async_copy(zin_hbm.at[srcv], rows, gsem).wait()
        pltpu.async_copy(rows, acc_sh.at[dstv], ssem, add=True)

        @pl.when(k + 2 < 80)
        def _():
            _load_idx(k + 2, srcv2, dstv2)
        pltpu.make_async_copy(rows, acc_sh.at[dstv], ssem).wait()

        @pl.when(k + 2 < 80)
        def _():
            pltpu.async_copy(zin_hbm.at[srcv2], rows, gsem)

    _load_idx(0, src_a0, dst_a0)
    pltpu.async_copy(zin_hbm.at[src_a0], rows_a, gsem_a)
    _load_idx(1, src_b0, dst_b0)
    pltpu.async_copy(zin_hbm.at[src_b0], rows_b, gsem_b)

    def _body(t, carry):
        k = 4 * t
        _half(k, src_a0, dst_a0, src_a1, dst_a1, rows_a, gsem_a, ssem_a)
        _half(k + 1, src_b0, dst_b0, src_b1, dst_b1, rows_b, gsem_b, ssem_b)
        _half(k + 2, src_a1, dst_a1, src_a0, dst_a0, rows_a, gsem_a, ssem_a)
        _half(k + 3, src_b1, dst_b1, src_b0, dst_b0, rows_b, gsem_b, ssem_b)
        return carry
    lax.fori_loop(0, 20, _body, 0)
    plsc.subcore_barrier()
    pltpu.sync_copy(acc_sh.at[pl.ds(s * RPT, RPT)],
                    zout_hbm.at[pl.ds(c * NP + s * RPT, RPT)])


def _conv_call(eidx, zin):
    return pl.kernel(
        _conv_body,
        out_type=jax.ShapeDtypeStruct((2 * NP, 128), jnp.float32),
        mesh=_MESH,
        compiler_params=_SC_PARAMS,
        scratch_types=[
            pltpu.VMEM_SHARED((NP, 128), jnp.float32),
            pltpu.VMEM((128,), jnp.int32),
            pltpu.VMEM((128,), jnp.int32),
            pltpu.VMEM((128,), jnp.int32),
            pltpu.VMEM((128,), jnp.int32),
            pltpu.VMEM((128,), jnp.int32),
            pltpu.VMEM((128,), jnp.int32),
            pltpu.VMEM((128,), jnp.int32),
            pltpu.VMEM((128,), jnp.int32),
            pltpu.VMEM((128, 128), jnp.float32),
            pltpu.VMEM((128, 128), jnp.float32),
            pltpu.SemaphoreType.DMA,
            pltpu.SemaphoreType.DMA,
            pltpu.SemaphoreType.DMA,
            pltpu.SemaphoreType.DMA,
        ],
    )(eidx, zin)


def _scale_body(z_ref, d2_ref, o_ref):
    o_ref[...] = z_ref[...] * d2_ref[...]


def _scale_call(z2, d2p):
    return pl.pallas_call(
        _scale_body,
        grid=(32,),
        in_specs=[
            pl.BlockSpec((RPT, 128), lambda j: (j, 0)),
            pl.BlockSpec((RPT, 1), lambda j: (j % 16, 0)),
        ],
        out_specs=pl.BlockSpec((RPT, 128), lambda j: (j, 0)),
        out_shape=jax.ShapeDtypeStruct((2 * NP, 128), jnp.float32),
    )(z2, d2p)


def _dot_body(ei_hbm, z4_hbm, d_hbm, out_hbm, dv, e2_a, e2_b, e2_c,
              ai_a, bi_a, ai_b, bi_b, ai_c, bi_c, rows_aa, rows_ab,
              rows_ba, rows_bb, rows_ca, rows_cb,
              ob_a, ob_b, ob_c, ga1, ga2, gb1, gb2, gc1, gc2,
              oa, ob_, oc):
    c = lax.axis_index("c")
    s = lax.axis_index("s")
    ebase = s * 10240
    roff = c * NP

    pltpu.sync_copy(d_hbm, dv)
    lane = lax.iota(jnp.int32, 16)

    def _issue(k, e2, ai, bi, ra, rb, s1, s2, outbuf, osem):
        pltpu.sync_copy(ei_hbm.at[:, pl.ds(ebase + k * 128, 128)], e2)
        for j in range(8):
            sl = pl.ds(j * 16, 16)
            ai[sl] = e2[0, sl] + roff
            bi[sl] = e2[1, sl] + roff
        pltpu.async_copy(z4_hbm.at[ai], ra, s1)
        pltpu.async_copy(z4_hbm.at[bi], rb, s2)

    def _compute(k, e2, ai, bi, ra, rb, s1, s2, outbuf, osem):
        pltpu.make_async_copy(z4_hbm.at[ai], ra, s1).wait()
        pltpu.make_async_copy(z4_hbm.at[bi], rb, s2).wait()

        @pl.when(k >= 3)
        def _():
            pltpu.make_async_copy(
                outbuf, out_hbm.at[c, pl.ds(ebase + (k - 3) * 128, 128)],
                osem).wait()
        for g in range(8):
            rowidx = g * 16 + lane

            def _cj(jo, accs):
                j0 = jo * 8
                out = []
                for jj in range(8):
                    cj = (jnp.full((16,), jj, jnp.int32) + j0 + lane) & 127
                    av = plsc.load_gather(ra, [rowidx, cj])
                    bv = plsc.load_gather(rb, [rowidx, cj])
                    out.append(accs[jj] + av * bv)
                return tuple(out)
            accs = lax.fori_loop(0, 16, _cj, (_Z16(),) * 8)
            res = (((accs[0] + accs[1]) + (accs[2] + accs[3]))
                   + ((accs[4] + accs[5]) + (accs[6] + accs[7])))
            sl = pl.ds(g * 16, 16)
            da = plsc.load_gather(dv, [e2[0, sl]])
            db = plsc.load_gather(dv, [e2[1, sl]])
            outbuf[sl] = res * da * db
        pltpu.async_copy(outbuf, out_hbm.at[c, pl.ds(ebase + k * 128, 128)],
                         osem)

    sets = ((e2_a, ai_a, bi_a, rows_aa, rows_ab, ga1, ga2, ob_a, oa),
            (e2_b, ai_b, bi_b, rows_ba, rows_bb, gb1, gb2, ob_b, ob_),
            (e2_c, ai_c, bi_c, rows_ca, rows_cb, gc1, gc2, ob_c, oc))
    _issue(0, *sets[0])
    _issue(1, *sets[1])
    _issue(2, *sets[2])

    def _body(t, carry):
        k = 3 * t
        for q in range(3):
            _compute(k + q, *sets[q])

            @pl.when(k + q + 3 < 80)
            def _():
                _issue(k + q + 3, *sets[q])
        return carry
    lax.fori_loop(0, 26, _body, 0)
    _compute(78, *sets[0])
    _compute(79, *sets[1])
    pltpu.make_async_copy(ob_a, out_hbm.at[c, pl.ds(ebase + 78 * 128, 128)],
                          oa).wait()
    pltpu.make_async_copy(ob_b, out_hbm.at[c, pl.ds(ebase + 79 * 128, 128)],
                          ob_).wait()
    pltpu.make_async_copy(ob_c, out_hbm.at[c, pl.ds(ebase + 77 * 128, 128)],
                          oc).wait()


_dot_call = functools.partial(
    pl.kernel,
    out_type=jax.ShapeDtypeStruct((NC, EPAD), jnp.float32),
    mesh=_MESH,
    compiler_params=_SC_PARAMS,
    scratch_types=[
        pltpu.VMEM((NP,), jnp.float32),
        pltpu.VMEM((2, 128), jnp.int32),
        pltpu.VMEM((2, 128), jnp.int32),
        pltpu.VMEM((2, 128), jnp.int32),
        pltpu.VMEM((128,), jnp.int32),
        pltpu.VMEM((128,), jnp.int32),
        pltpu.VMEM((128,), jnp.int32),
        pltpu.VMEM((128,), jnp.int32),
        pltpu.VMEM((128,), jnp.int32),
        pltpu.VMEM((128,), jnp.int32),
        pltpu.VMEM((128, 128), jnp.float32),
        pltpu.VMEM((128, 128), jnp.float32),
        pltpu.VMEM((128, 128), jnp.float32),
        pltpu.VMEM((128, 128), jnp.float32),
        pltpu.VMEM((128, 128), jnp.float32),
        pltpu.VMEM((128, 128), jnp.float32),
        pltpu.VMEM((128,), jnp.float32),
        pltpu.VMEM((128,), jnp.float32),
        pltpu.VMEM((128,), jnp.float32),
        pltpu.SemaphoreType.DMA,
        pltpu.SemaphoreType.DMA,
        pltpu.SemaphoreType.DMA,
        pltpu.SemaphoreType.DMA,
        pltpu.SemaphoreType.DMA,
        pltpu.SemaphoreType.DMA,
        pltpu.SemaphoreType.DMA,
        pltpu.SemaphoreType.DMA,
        pltpu.SemaphoreType.DMA,
    ],
)(_dot_body)


def _comb_body(p_ref, o_ref):
    o_ref[...] = p_ref[0, :] + p_ref[1, :]


def _comb_call(partials):
    return pl.pallas_call(
        _comb_body,
        grid=(80,),
        in_specs=[pl.BlockSpec((2, 2048), lambda j: (0, j))],
        out_specs=pl.BlockSpec((2048,), lambda j: (j,)),
        out_shape=jax.ShapeDtypeStruct((EPAD,), jnp.float32),
    )(partials)


def kernel(x_input, edge_index_input, pos_edge_index, W, b):
    padv = (N + jnp.arange(EPAD - E, dtype=jnp.int32) % (NP - N))
    epad = jnp.concatenate(
        [pos_edge_index, jnp.stack([padv, padv])], axis=1)
    eipad = jnp.concatenate(
        [edge_index_input, jnp.stack([padv, padv])], axis=1)
    b2 = b.reshape(2, 1, 128)

    degp = _deg_call(epad)
    x1h3, dcol, d2col = _linear_call(x_input, W, b2, degp.reshape(2, NP, 1))
    x1h = x1h3.reshape(2 * NP, 128)

    z2 = _conv_call(epad, x1h)
    z3 = _scale_call(z2, d2col)
    z4 = _conv_call(epad, z3)
    partials = _dot_call(eipad, z4, dcol.reshape(NP))
    logits = _comb_call(partials)
    return logits[:E]
